# baseline (speedup 1.0000x reference)
import functools

import jax
import jax.numpy as jnp
from jax import lax
from jax.experimental import pallas as pl
from jax.experimental.pallas import tpu as pltpu
from jax.experimental.pallas import tpu_sc as plsc

N = 10000
E = 320000
D = 128
H = 128

NC = 2
NS = 16
NW = NC * NS
CS = 128
CHB = 8
NB = 10
EW = NB * CHB * CS
EP = NW * EW
RPT = 632
NP = RPT * NS

_mesh = plsc.VectorSubcoreMesh(
    core_axis_name="c", subcore_axis_name="s", num_cores=NC, num_subcores=NS
)


@functools.partial(
    pl.kernel,
    out_type=jax.ShapeDtypeStruct((NC, NP, D), jnp.float32),
    mesh=_mesh,
    scratch_types=[
        pltpu.VMEM((2, CHB, CS), jnp.int32),
        pltpu.VMEM((2, CHB, CS), jnp.int32),
        [pltpu.VMEM((CS, D), jnp.float32) for _ in range(2)],
        [pltpu.SemaphoreType.DMA for _ in range(2)],
        [pltpu.SemaphoreType.DMA for _ in range(2)],
        pltpu.VMEM_SHARED((NP, D), jnp.float32),
    ],
)
def _sc_agg(x_hbm, src_hbm, dst_hbm, zeros_hbm, out_hbm,
            src_v, dst_v, rows, gsem, ssem, acc):
    c = lax.axis_index("c")
    s = lax.axis_index("s")
    wid = s * NC + c
    r0 = s * RPT

    pltpu.sync_copy(zeros_hbm.at[pl.ds(r0, RPT)], acc.at[pl.ds(r0, RPT)])
    plsc.subcore_barrier()

    def pair(par, jo, first):
        handles = []
        for k in range(2):
            j = jo + k
            if not first:
                pltpu.make_async_copy(
                    rows[k], acc.at[dst_v.at[par, j]], ssem[k]).wait()
            handles.append(
                pltpu.async_copy(x_hbm.at[src_v.at[par, j]], rows[k], gsem[k]))
        for k in range(2):
            handles[k].wait()
            pltpu.make_async_copy(
                rows[k], acc.at[dst_v.at[par, jo + k]], ssem[k]).start(add=True)

    def stage(b, par):
        pltpu.sync_copy(src_hbm.at[wid, b], src_v.at[par])
        pltpu.sync_copy(dst_hbm.at[wid, b], dst_v.at[par])

    stage(0, 0)
    pair(0, 0, True)
    for jo in range(2, CHB, 2):
        pair(0, jo, False)

    def body(b, carry):
        par = b % 2
        stage(b, par)
        for jo in range(0, CHB, 2):
            pair(par, jo, False)
        return carry

    lax.fori_loop(1, NB, body, 0)

    for k in range(2):
        pltpu.make_async_copy(
            rows[k], acc.at[dst_v.at[1, CHB - 2 + k]], ssem[k]).wait()

    plsc.subcore_barrier()
    pltpu.sync_copy(acc.at[pl.ds(r0, RPT)], out_hbm.at[c, pl.ds(r0, RPT)])


BLK = 1000


def _mlp_body(eps_ref, x_ref, agg_ref, wa_ref, ba_ref, wb_ref, bb_ref, o_ref):
    upd = (1.0 + eps_ref[0, 0]) * x_ref[...] + agg_ref[0] + agg_ref[1]
    h = jnp.maximum(
        jnp.dot(upd, wa_ref[...], preferred_element_type=jnp.float32)
        + ba_ref[...], 0.0)
    o_ref[...] = (
        jnp.dot(h, wb_ref[...], preferred_element_type=jnp.float32)
        + bb_ref[...])


def _mlp2_body(eps_ref, x_ref, agg_ref, wa_ref, ba_ref, wb_ref, bb_ref,
               wo_ref, bo_ref, o_ref):
    upd = (1.0 + eps_ref[0, 0]) * x_ref[...] + agg_ref[0] + agg_ref[1]
    h = jnp.maximum(
        jnp.dot(upd, wa_ref[...], preferred_element_type=jnp.float32)
        + ba_ref[...], 0.0)
    y = (jnp.dot(h, wb_ref[...], preferred_element_type=jnp.float32)
         + bb_ref[...])
    o_ref[...] = (
        jnp.dot(y, wo_ref[...], preferred_element_type=jnp.float32)
        + bo_ref[...])


def _w_spec(r, c_):
    return pl.BlockSpec((r, c_), lambda i: (0, 0))


def _mlp(eps, x, agg, wa, ba, wb, bb):
    return pl.pallas_call(
        _mlp_body,
        grid=(N // BLK,),
        in_specs=[
            pl.BlockSpec(memory_space=pltpu.SMEM),
            pl.BlockSpec((BLK, D), lambda i: (i, 0)),
            pl.BlockSpec((NC, BLK, D), lambda i: (0, i, 0)),
            _w_spec(D, H), _w_spec(1, H), _w_spec(H, H), _w_spec(1, H),
        ],
        out_specs=pl.BlockSpec((BLK, H), lambda i: (i, 0)),
        out_shape=jax.ShapeDtypeStruct((N, H), jnp.float32),
    )(eps, x, agg, wa, ba, wb, bb)


def _mlp2(eps, x, agg, wa, ba, wb, bb, wo, bo):
    return pl.pallas_call(
        _mlp2_body,
        grid=(N // BLK,),
        in_specs=[
            pl.BlockSpec(memory_space=pltpu.SMEM),
            pl.BlockSpec((BLK, H), lambda i: (i, 0)),
            pl.BlockSpec((NC, BLK, H), lambda i: (0, i, 0)),
            _w_spec(H, H), _w_spec(1, H), _w_spec(H, H), _w_spec(1, H),
            _w_spec(H, D), _w_spec(1, D),
        ],
        out_specs=pl.BlockSpec((BLK, D), lambda i: (i, 0)),
        out_shape=jax.ShapeDtypeStruct((N, D), jnp.float32),
    )(eps, x, agg, wa, ba, wb, bb, wo, bo)


def kernel(node_features, edge_index, eps1, W1a, b1a, W1b, b1b,
           eps2, W2a, b2a, W2b, b2b, Wout, bout):
    pad = EP - E
    pad_iota = jax.lax.iota(jnp.int32, pad)
    src = jnp.concatenate(
        [edge_index[0], pad_iota % N]).reshape(NW, NB, CHB, CS)
    dst = jnp.concatenate(
        [edge_index[1], N + pad_iota % (NP - N)]).reshape(NW, NB, CHB, CS)
    zeros = jnp.zeros((NP, D), jnp.float32)
    eps1r = jnp.reshape(eps1, (1, 1))
    eps2r = jnp.reshape(eps2, (1, 1))

    agg1 = _sc_agg(node_features, src, dst, zeros)[:, :N]
    x1 = _mlp(eps1r, node_features, agg1, W1a, b1a.reshape(1, H),
              W1b, b1b.reshape(1, H))
    agg2 = _sc_agg(x1, src, dst, zeros)[:, :N]
    return _mlp2(eps2r, x1, agg2, W2a, b2a.reshape(1, H),
                 W2b, b2b.reshape(1, H), Wout, bout.reshape(1, D))


# speedup vs baseline: 1.0775x; 1.0775x over previous
import functools

import jax
import jax.numpy as jnp
from jax import lax
from jax.experimental import pallas as pl
from jax.experimental.pallas import tpu as pltpu
from jax.experimental.pallas import tpu_sc as plsc

N = 10000
E = 320000
D = 128
H = 128

NC = 2
NS = 16
NW = NC * NS
CS = 64
CHB = 8
NB = 20
NQ = CHB // 4
EW = NB * CHB * CS
EP = NW * EW
RPT = 632
NP = RPT * NS

_mesh = plsc.VectorSubcoreMesh(
    core_axis_name="c", subcore_axis_name="s", num_cores=NC, num_subcores=NS
)


@functools.partial(
    pl.kernel,
    out_type=jax.ShapeDtypeStruct((NC, NP, D), jnp.float32),
    mesh=_mesh,
    scratch_types=[
        pltpu.VMEM((2, CHB, CS), jnp.int32),
        pltpu.VMEM((2, CHB, CS), jnp.int32),
        [pltpu.VMEM((CS, D), jnp.float32) for _ in range(4)],
        [pltpu.SemaphoreType.DMA for _ in range(4)],
        [pltpu.SemaphoreType.DMA for _ in range(4)],
        pltpu.VMEM_SHARED((NP, D), jnp.float32),
    ],
)
def _sc_agg(x_hbm, src_hbm, dst_hbm, zeros_hbm, out_hbm,
            src_v, dst_v, rows, gsem, ssem, acc):
    c = lax.axis_index("c")
    s = lax.axis_index("s")
    wid = s * NC + c
    r0 = s * RPT

    pltpu.sync_copy(zeros_hbm.at[pl.ds(r0, RPT)], acc.at[pl.ds(r0, RPT)])
    plsc.subcore_barrier()

    def quad(par, jo, first):
        handles = []
        for k in range(4):
            j = jo + k
            if not first:
                pltpu.make_async_copy(
                    rows[k], acc.at[dst_v.at[par, j]], ssem[k]).wait()
            handles.append(
                pltpu.async_copy(x_hbm.at[src_v.at[par, j]], rows[k], gsem[k]))
        for k in range(4):
            handles[k].wait()
            pltpu.make_async_copy(
                rows[k], acc.at[dst_v.at[par, jo + k]], ssem[k]).start(add=True)

    def stage(b, par):
        pltpu.sync_copy(src_hbm.at[wid, b], src_v.at[par])
        pltpu.sync_copy(dst_hbm.at[wid, b], dst_v.at[par])

    stage(0, 0)
    quad(0, 0, True)
    quad(0, 4, False)

    def body(b, carry):
        par = b % 2
        stage(b, par)
        quad(par, 0, False)
        quad(par, 4, False)
        return carry

    lax.fori_loop(1, NB, body, 0)

    for k in range(4):
        pltpu.make_async_copy(
            rows[k], acc.at[dst_v.at[1, 4 + k]], ssem[k]).wait()

    plsc.subcore_barrier()
    pltpu.sync_copy(acc.at[pl.ds(r0, RPT)], out_hbm.at[c, pl.ds(r0, RPT)])


BLK = 1000


def _mlp_body(eps_ref, x_ref, agg_ref, wa_ref, ba_ref, wb_ref, bb_ref, o_ref):
    upd = (1.0 + eps_ref[0, 0]) * x_ref[...] + agg_ref[0] + agg_ref[1]
    h = jnp.maximum(
        jnp.dot(upd, wa_ref[...], preferred_element_type=jnp.float32)
        + ba_ref[...], 0.0)
    o_ref[...] = (
        jnp.dot(h, wb_ref[...], preferred_element_type=jnp.float32)
        + bb_ref[...])


def _mlp2_body(eps_ref, x_ref, agg_ref, wa_ref, ba_ref, wb_ref, bb_ref,
               wo_ref, bo_ref, o_ref):
    upd = (1.0 + eps_ref[0, 0]) * x_ref[...] + agg_ref[0] + agg_ref[1]
    h = jnp.maximum(
        jnp.dot(upd, wa_ref[...], preferred_element_type=jnp.float32)
        + ba_ref[...], 0.0)
    y = (jnp.dot(h, wb_ref[...], preferred_element_type=jnp.float32)
         + bb_ref[...])
    o_ref[...] = (
        jnp.dot(y, wo_ref[...], preferred_element_type=jnp.float32)
        + bo_ref[...])


def _w_spec(r, c_):
    return pl.BlockSpec((r, c_), lambda i: (0, 0))


def _mlp(eps, x, agg, wa, ba, wb, bb):
    return pl.pallas_call(
        _mlp_body,
        grid=(N // BLK,),
        in_specs=[
            pl.BlockSpec(memory_space=pltpu.SMEM),
            pl.BlockSpec((BLK, D), lambda i: (i, 0)),
            pl.BlockSpec((NC, BLK, D), lambda i: (0, i, 0)),
            _w_spec(D, H), _w_spec(1, H), _w_spec(H, H), _w_spec(1, H),
        ],
        out_specs=pl.BlockSpec((BLK, H), lambda i: (i, 0)),
        out_shape=jax.ShapeDtypeStruct((N, H), jnp.float32),
    )(eps, x, agg, wa, ba, wb, bb)


def _mlp2(eps, x, agg, wa, ba, wb, bb, wo, bo):
    return pl.pallas_call(
        _mlp2_body,
        grid=(N // BLK,),
        in_specs=[
            pl.BlockSpec(memory_space=pltpu.SMEM),
            pl.BlockSpec((BLK, H), lambda i: (i, 0)),
            pl.BlockSpec((NC, BLK, H), lambda i: (0, i, 0)),
            _w_spec(H, H), _w_spec(1, H), _w_spec(H, H), _w_spec(1, H),
            _w_spec(H, D), _w_spec(1, D),
        ],
        out_specs=pl.BlockSpec((BLK, D), lambda i: (i, 0)),
        out_shape=jax.ShapeDtypeStruct((N, D), jnp.float32),
    )(eps, x, agg, wa, ba, wb, bb, wo, bo)


def kernel(node_features, edge_index, eps1, W1a, b1a, W1b, b1b,
           eps2, W2a, b2a, W2b, b2b, Wout, bout):
    pad = EP - E
    pad_iota = jax.lax.iota(jnp.int32, pad)
    src = jnp.concatenate(
        [edge_index[0], pad_iota % N]).reshape(NW, NB, CHB, CS)
    dst = jnp.concatenate(
        [edge_index[1], N + pad_iota % (NP - N)]).reshape(NW, NB, CHB, CS)
    zeros = jnp.zeros((NP, D), jnp.float32)
    eps1r = jnp.reshape(eps1, (1, 1))
    eps2r = jnp.reshape(eps2, (1, 1))

    agg1 = _sc_agg(node_features, src, dst, zeros)
    x1 = _mlp(eps1r, node_features, agg1, W1a, b1a.reshape(1, H),
              W1b, b1b.reshape(1, H))
    agg2 = _sc_agg(x1, src, dst, zeros)
    return _mlp2(eps2r, x1, agg2, W2a, b2a.reshape(1, H),
                 W2b, b2b.reshape(1, H), Wout, bout.reshape(1, D))


# speedup vs baseline: 1.1788x; 1.0940x over previous
import functools

import jax
import jax.numpy as jnp
from jax import lax
from jax.experimental import pallas as pl
from jax.experimental.pallas import tpu as pltpu
from jax.experimental.pallas import tpu_sc as plsc

N = 10000
E = 320000
D = 128
H = 128

NC = 2
NS = 16
NW = NC * NS
CS = 128
SS = CS // 2
CHB = 8
NB = 10
EW = NB * CHB * CS
EP = NW * EW
RPT = 632
NP = RPT * NS

_mesh = plsc.VectorSubcoreMesh(
    core_axis_name="c", subcore_axis_name="s", num_cores=NC, num_subcores=NS
)


@functools.partial(
    pl.kernel,
    out_type=jax.ShapeDtypeStruct((NC, NP, D), jnp.float32),
    mesh=_mesh,
    scratch_types=[
        pltpu.VMEM((2, CHB, CS), jnp.int32),
        pltpu.VMEM((2, 2 * CHB, SS), jnp.int32),
        [pltpu.VMEM((CS, D), jnp.float32) for _ in range(2)],
        [pltpu.SemaphoreType.DMA for _ in range(2)],
        [pltpu.SemaphoreType.DMA for _ in range(4)],
        pltpu.VMEM_SHARED((NP, D), jnp.float32),
    ],
)
def _sc_agg(x_hbm, src_hbm, dst_hbm, zeros_hbm, out_hbm,
            src_v, dst_v, rows, gsem, ssem, acc):
    c = lax.axis_index("c")
    s = lax.axis_index("s")
    wid = s * NC + c
    r0 = s * RPT

    pltpu.sync_copy(zeros_hbm.at[pl.ds(r0, RPT)], acc.at[pl.ds(r0, RPT)])
    plsc.subcore_barrier()

    def pair(par, jo, first):
        handles = []
        for k in range(2):
            j = jo + k
            if not first:
                for h in range(2):
                    pltpu.make_async_copy(
                        rows[k].at[pl.ds(SS * h, SS)],
                        acc.at[dst_v.at[par, 2 * j + h]],
                        ssem[2 * k + h]).wait()
            handles.append(
                pltpu.async_copy(x_hbm.at[src_v.at[par, j]], rows[k], gsem[k]))
        for k in range(2):
            j = jo + k
            handles[k].wait()
            for h in range(2):
                pltpu.make_async_copy(
                    rows[k].at[pl.ds(SS * h, SS)],
                    acc.at[dst_v.at[par, 2 * j + h]],
                    ssem[2 * k + h]).start(add=True)

    def stage(b, par):
        pltpu.sync_copy(src_hbm.at[wid, b], src_v.at[par])
        pltpu.sync_copy(dst_hbm.at[wid, b], dst_v.at[par])

    stage(0, 0)
    pair(0, 0, True)
    for jo in range(2, CHB, 2):
        pair(0, jo, False)

    def body(b, carry):
        par = b % 2
        stage(b, par)
        for jo in range(0, CHB, 2):
            pair(par, jo, False)
        return carry

    lax.fori_loop(1, NB, body, 0)

    for k in range(2):
        for h in range(2):
            pltpu.make_async_copy(
                rows[k].at[pl.ds(SS * h, SS)],
                acc.at[dst_v.at[1, 2 * (CHB - 2 + k) + h]],
                ssem[2 * k + h]).wait()

    plsc.subcore_barrier()
    pltpu.sync_copy(acc.at[pl.ds(r0, RPT)], out_hbm.at[c, pl.ds(r0, RPT)])


BLK = 1000


def _mlp_body(eps_ref, x_ref, agg_ref, wa_ref, ba_ref, wb_ref, bb_ref, o_ref):
    upd = (1.0 + eps_ref[0, 0]) * x_ref[...] + agg_ref[0] + agg_ref[1]
    h = jnp.maximum(
        jnp.dot(upd, wa_ref[...], preferred_element_type=jnp.float32)
        + ba_ref[...], 0.0)
    o_ref[...] = (
        jnp.dot(h, wb_ref[...], preferred_element_type=jnp.float32)
        + bb_ref[...])


def _mlp2_body(eps_ref, x_ref, agg_ref, wa_ref, ba_ref, wb_ref, bb_ref,
               wo_ref, bo_ref, o_ref):
    upd = (1.0 + eps_ref[0, 0]) * x_ref[...] + agg_ref[0] + agg_ref[1]
    h = jnp.maximum(
        jnp.dot(upd, wa_ref[...], preferred_element_type=jnp.float32)
        + ba_ref[...], 0.0)
    y = (jnp.dot(h, wb_ref[...], preferred_element_type=jnp.float32)
         + bb_ref[...])
    o_ref[...] = (
        jnp.dot(y, wo_ref[...], preferred_element_type=jnp.float32)
        + bo_ref[...])


def _w_spec(r, c_):
    return pl.BlockSpec((r, c_), lambda i: (0, 0))


def _mlp(eps, x, agg, wa, ba, wb, bb):
    return pl.pallas_call(
        _mlp_body,
        grid=(N // BLK,),
        in_specs=[
            pl.BlockSpec(memory_space=pltpu.SMEM),
            pl.BlockSpec((BLK, D), lambda i: (i, 0)),
            pl.BlockSpec((NC, BLK, D), lambda i: (0, i, 0)),
            _w_spec(D, H), _w_spec(1, H), _w_spec(H, H), _w_spec(1, H),
        ],
        out_specs=pl.BlockSpec((BLK, H), lambda i: (i, 0)),
        out_shape=jax.ShapeDtypeStruct((N, H), jnp.float32),
    )(eps, x, agg, wa, ba, wb, bb)


def _mlp2(eps, x, agg, wa, ba, wb, bb, wo, bo):
    return pl.pallas_call(
        _mlp2_body,
        grid=(N // BLK,),
        in_specs=[
            pl.BlockSpec(memory_space=pltpu.SMEM),
            pl.BlockSpec((BLK, H), lambda i: (i, 0)),
            pl.BlockSpec((NC, BLK, H), lambda i: (0, i, 0)),
            _w_spec(H, H), _w_spec(1, H), _w_spec(H, H), _w_spec(1, H),
            _w_spec(H, D), _w_spec(1, D),
        ],
        out_specs=pl.BlockSpec((BLK, D), lambda i: (i, 0)),
        out_shape=jax.ShapeDtypeStruct((N, D), jnp.float32),
    )(eps, x, agg, wa, ba, wb, bb, wo, bo)


def kernel(node_features, edge_index, eps1, W1a, b1a, W1b, b1b,
           eps2, W2a, b2a, W2b, b2b, Wout, bout):
    pad = EP - E
    pad_iota = jax.lax.iota(jnp.int32, pad)
    src = jnp.concatenate(
        [edge_index[0], pad_iota % N]).reshape(NW, NB, CHB, CS)
    dst = jnp.concatenate(
        [edge_index[1], N + pad_iota % (NP - N)]).reshape(NW, NB, 2 * CHB, SS)
    zeros = jnp.zeros((NP, D), jnp.float32)
    eps1r = jnp.reshape(eps1, (1, 1))
    eps2r = jnp.reshape(eps2, (1, 1))

    agg1 = _sc_agg(node_features, src, dst, zeros)
    x1 = _mlp(eps1r, node_features, agg1, W1a, b1a.reshape(1, H),
              W1b, b1b.reshape(1, H))
    agg2 = _sc_agg(x1, src, dst, zeros)
    return _mlp2(eps2r, x1, agg2, W2a, b2a.reshape(1, H),
                 W2b, b2b.reshape(1, H), Wout, bout.reshape(1, D))
